# baseline (speedup 1.0000x reference)
import jax
import jax.numpy as jnp
from jax import lax
from jax.experimental import pallas as pl
from jax.experimental.pallas import tpu as pltpu
from jax.experimental.pallas import tpu_sc as plsc

N = 10000
E = 320000
D = 128

NC = 2
NS = 16
NW = NC * NS
EDGES_PER_TILE = E // NW
CHUNK = 128
NCHUNK = 79
EPT_PAD = NCHUNK * CHUNK
ROWS_PAD = 10240
ZROWS = ROWS_PAD // NS
WB = 624

BLK = 2000
GRID = N // BLK


def _mlp_body(x_ref, w1_ref, b1_ref, w2_ref, b2_ref, o_ref):
    x = x_ref[...]
    h = jnp.dot(x, w1_ref[...], preferred_element_type=jnp.float32) + b1_ref[...]
    h = jnp.maximum(h, 0.0)
    h = jnp.dot(h, w2_ref[...], preferred_element_type=jnp.float32) + b2_ref[...]
    h = h * jax.nn.sigmoid(h)
    mu = jnp.mean(h, axis=1, keepdims=True)
    d = h - mu
    var = jnp.mean(d * d, axis=1, keepdims=True)
    o_ref[...] = d * lax.rsqrt(var + 1e-5)


_mlp = pl.pallas_call(
    _mlp_body,
    grid=(GRID,),
    in_specs=[
        pl.BlockSpec((BLK, D), lambda i: (i, 0)),
        pl.BlockSpec((D, D), lambda i: (0, 0)),
        pl.BlockSpec((1, D), lambda i: (0, 0)),
        pl.BlockSpec((D, D), lambda i: (0, 0)),
        pl.BlockSpec((1, D), lambda i: (0, 0)),
    ],
    out_specs=pl.BlockSpec((BLK, D), lambda i: (i, 0)),
    out_shape=jax.ShapeDtypeStruct((N, D), jnp.float32),
)


def _combine_body(a_ref, c_ref, hp_ref, o_ref):
    s = a_ref[0] + a_ref[1]
    cnt = jnp.sum(c_ref[...], axis=1).reshape(-1, 1)
    o_ref[...] = s / jnp.maximum(cnt, 1.0) + hp_ref[...]


_combine = pl.pallas_call(
    _combine_body,
    grid=(GRID,),
    in_specs=[
        pl.BlockSpec((2, BLK, D), lambda i: (0, i, 0)),
        pl.BlockSpec((BLK, NW), lambda i: (i, 0)),
        pl.BlockSpec((BLK, D), lambda i: (i, 0)),
    ],
    out_specs=pl.BlockSpec((BLK, D), lambda i: (i, 0)),
    out_shape=jax.ShapeDtypeStruct((N, D), jnp.float32),
)


def _agg_body(z_hbm, src_hbm, dst_hbm, out_hbm, sidx, didx, gidx, widx,
              rows, acc_sp, sem):
    c = lax.axis_index("c")
    s = lax.axis_index("s")
    wid = c * NS + s

    def _preload(g, _):
        pltpu.sync_copy(src_hbm.at[wid, pl.ds(g * 8, 8)],
                        sidx.at[pl.ds(g * 8, 8)])
        pltpu.sync_copy(dst_hbm.at[wid, pl.ds(g * 8, 8)],
                        didx.at[pl.ds(g * 8, 8)])
        return 0

    lax.fori_loop(0, NCHUNK // 8, _preload, 0)
    _tail = (NCHUNK // 8) * 8
    pltpu.sync_copy(src_hbm.at[wid, pl.ds(_tail, NCHUNK - _tail)],
                    sidx.at[pl.ds(_tail, NCHUNK - _tail)])
    pltpu.sync_copy(dst_hbm.at[wid, pl.ds(_tail, NCHUNK - _tail)],
                    didx.at[pl.ds(_tail, NCHUNK - _tail)])

    zeros16 = jnp.zeros((16,), jnp.float32)

    def _zero_row(r, _):
        for cc in range(D // 16):
            rows[r, pl.ds(cc * 16, 16)] = zeros16
        return 0

    lax.fori_loop(0, CHUNK, _zero_row, 0)
    for k in range(ZROWS // CHUNK):
        pltpu.sync_copy(rows, acc_sp.at[pl.ds(s * ZROWS + k * CHUNK, CHUNK)])
    plsc.subcore_barrier()

    def _chunk(j, _):
        for g in range(CHUNK // 16):
            gidx[pl.ds(g * 16, 16)] = sidx[j, pl.ds(g * 16, 16)]
            widx[pl.ds(g * 16, 16)] = didx[j, pl.ds(g * 16, 16)]
        pltpu.async_copy(z_hbm.at[gidx], rows, sem).wait()
        pltpu.sync_copy(rows, acc_sp.at[widx], add=True)
        return 0

    lax.fori_loop(0, NCHUNK, _chunk, 0)
    plsc.subcore_barrier()

    pltpu.sync_copy(acc_sp.at[pl.ds(s * WB, WB)],
                    out_hbm.at[c, pl.ds(s * WB, WB)])

    @pl.when(s == NS - 1)
    def _():
        pltpu.sync_copy(acc_sp.at[pl.ds(NS * WB, N - NS * WB)],
                        out_hbm.at[c, pl.ds(NS * WB, N - NS * WB)])


_aggregate = pl.kernel(
    _agg_body,
    out_type=jax.ShapeDtypeStruct((NC, N, D), jnp.float32),
    mesh=plsc.VectorSubcoreMesh(core_axis_name="c", subcore_axis_name="s"),
    scratch_types=[
        pltpu.VMEM((NCHUNK, CHUNK), jnp.int32),
        pltpu.VMEM((NCHUNK, CHUNK), jnp.int32),
        pltpu.VMEM((CHUNK,), jnp.int32),
        pltpu.VMEM((CHUNK,), jnp.int32),
        pltpu.VMEM((CHUNK, D), jnp.float32),
        pltpu.VMEM_SHARED((ROWS_PAD, D), jnp.float32),
        pltpu.SemaphoreType.DMA,
    ],
    compiler_params=pltpu.CompilerParams(needs_layout_passes=False),
)


def _cnt_body(dst_hbm, cnt_hbm, didx, cnt_v):
    c = lax.axis_index("c")
    s = lax.axis_index("s")
    wid = c * NS + s

    zeros16 = jnp.zeros((16,), jnp.float32)

    def _zero_cnt(r, _):
        cnt_v[0, pl.ds(r * 16, 16)] = zeros16
        return 0

    lax.fori_loop(0, N // 16, _zero_cnt, 0)
    pltpu.sync_copy(dst_hbm.at[wid], didx)

    ones16 = jnp.full((16,), 1.0, jnp.float32)

    def _group(g, _):
        idx = didx[0, pl.ds(g * 16, 16)]
        plsc.addupdate_scatter(cnt_v.at[0], [idx], ones16)
        return 0

    lax.fori_loop(0, EDGES_PER_TILE // 16, _group, 0)
    pltpu.sync_copy(cnt_v, cnt_hbm.at[wid])


_count_degrees = pl.kernel(
    _cnt_body,
    out_type=jax.ShapeDtypeStruct((NW, 1, N), jnp.float32),
    mesh=plsc.VectorSubcoreMesh(core_axis_name="c", subcore_axis_name="s"),
    scratch_types=[
        pltpu.VMEM((1, EDGES_PER_TILE), jnp.int32),
        pltpu.VMEM((1, N), jnp.float32),
    ],
    compiler_params=pltpu.CompilerParams(needs_layout_passes=False),
)


def kernel(x, edge_index, W1_0, b1_0, W2_0, b2_0, W1_1, b1_1, W2_1, b2_1,
           W1_2, b1_2, W2_2, b2_2):
    npad = EPT_PAD - EDGES_PER_TILE
    pad_src = jnp.zeros((NW, npad), jnp.int32)
    pad_dst = jnp.broadcast_to(
        N + (jnp.arange(npad, dtype=jnp.int32) % (ROWS_PAD - N)), (NW, npad))
    src = jnp.concatenate(
        [edge_index[0].reshape(NW, EDGES_PER_TILE), pad_src],
        axis=1).reshape(NW, NCHUNK, CHUNK)
    dst = jnp.concatenate(
        [edge_index[1].reshape(NW, EDGES_PER_TILE), pad_dst],
        axis=1).reshape(NW, NCHUNK, CHUNK)
    dstw = edge_index[1].reshape(NW, 1, EDGES_PER_TILE)
    params = [
        (W1_0, b1_0, W2_0, b2_0),
        (W1_1, b1_1, W2_1, b2_1),
        (W1_2, b1_2, W2_2, b2_2),
    ]
    cnt = _count_degrees(dstw).reshape(NW, N).T
    h = x
    for (w1, b1, w2, b2) in params:
        z = _mlp(h, w1, b1.reshape(1, D), w2, b2.reshape(1, D))
        acc = _aggregate(z, src, dst)
        h = _combine(acc, cnt, h)
    return h


# speedup vs baseline: 1.0458x; 1.0458x over previous
import jax
import jax.numpy as jnp
from jax import lax
from jax.experimental import pallas as pl
from jax.experimental.pallas import tpu as pltpu
from jax.experimental.pallas import tpu_sc as plsc

N = 10000
E = 320000
D = 128

NC = 2
NS = 16
NW = NC * NS
EDGES_PER_TILE = E // NW
CHUNK = 64
NCHUNK = 157
EPT_PAD = NCHUNK * CHUNK
ROWS_PAD = 10208
ZROWS = ROWS_PAD // NS
WB = 624

BLK = 2000
GRID = N // BLK


def _mlp_body(x_ref, w1_ref, b1_ref, w2_ref, b2_ref, o_ref):
    x = x_ref[...]
    h = jnp.dot(x, w1_ref[...], preferred_element_type=jnp.float32) + b1_ref[...]
    h = jnp.maximum(h, 0.0)
    h = jnp.dot(h, w2_ref[...], preferred_element_type=jnp.float32) + b2_ref[...]
    h = h * jax.nn.sigmoid(h)
    mu = jnp.mean(h, axis=1, keepdims=True)
    d = h - mu
    var = jnp.mean(d * d, axis=1, keepdims=True)
    o_ref[...] = d * lax.rsqrt(var + 1e-5)


_mlp = pl.pallas_call(
    _mlp_body,
    grid=(GRID,),
    in_specs=[
        pl.BlockSpec((BLK, D), lambda i: (i, 0)),
        pl.BlockSpec((D, D), lambda i: (0, 0)),
        pl.BlockSpec((1, D), lambda i: (0, 0)),
        pl.BlockSpec((D, D), lambda i: (0, 0)),
        pl.BlockSpec((1, D), lambda i: (0, 0)),
    ],
    out_specs=pl.BlockSpec((BLK, D), lambda i: (i, 0)),
    out_shape=jax.ShapeDtypeStruct((N, D), jnp.float32),
)


def _combine_body(a_ref, c_ref, hp_ref, o_ref):
    s = a_ref[0] + a_ref[1]
    cnt = jnp.sum(c_ref[...], axis=1).reshape(-1, 1)
    o_ref[...] = s / jnp.maximum(cnt, 1.0) + hp_ref[...]


_combine = pl.pallas_call(
    _combine_body,
    grid=(GRID,),
    in_specs=[
        pl.BlockSpec((2, BLK, D), lambda i: (0, i, 0)),
        pl.BlockSpec((BLK, NW), lambda i: (i, 0)),
        pl.BlockSpec((BLK, D), lambda i: (i, 0)),
    ],
    out_specs=pl.BlockSpec((BLK, D), lambda i: (i, 0)),
    out_shape=jax.ShapeDtypeStruct((N, D), jnp.float32),
)


def _agg_body(z_hbm, src_hbm, dst_hbm, out_hbm, sidx, didx, gidx, widx,
              rows, acc_sp, sem):
    c = lax.axis_index("c")
    s = lax.axis_index("s")
    wid = c * NS + s

    def _preload(g, _):
        pltpu.sync_copy(src_hbm.at[wid, pl.ds(g * 8, 8)],
                        sidx.at[pl.ds(g * 8, 8)])
        pltpu.sync_copy(dst_hbm.at[wid, pl.ds(g * 8, 8)],
                        didx.at[pl.ds(g * 8, 8)])
        return 0

    lax.fori_loop(0, NCHUNK // 8, _preload, 0)
    _tail = (NCHUNK // 8) * 8
    pltpu.sync_copy(src_hbm.at[wid, pl.ds(_tail, NCHUNK - _tail)],
                    sidx.at[pl.ds(_tail, NCHUNK - _tail)])
    pltpu.sync_copy(dst_hbm.at[wid, pl.ds(_tail, NCHUNK - _tail)],
                    didx.at[pl.ds(_tail, NCHUNK - _tail)])

    zeros16 = jnp.zeros((16,), jnp.float32)

    def _zero_row(r, _):
        for cc in range(D // 16):
            rows[r, pl.ds(cc * 16, 16)] = zeros16
        return 0

    lax.fori_loop(0, CHUNK, _zero_row, 0)
    for k in range(ZROWS // CHUNK):
        pltpu.sync_copy(rows, acc_sp.at[pl.ds(s * ZROWS + k * CHUNK, CHUNK)])
    if ZROWS % CHUNK:
        pltpu.sync_copy(rows, acc_sp.at[pl.ds(s * ZROWS + ZROWS - CHUNK,
                                              CHUNK)])
    plsc.subcore_barrier()

    def _chunk(j, _):
        for g in range(CHUNK // 16):
            gidx[pl.ds(g * 16, 16)] = sidx[j, pl.ds(g * 16, 16)]
            widx[pl.ds(g * 16, 16)] = didx[j, pl.ds(g * 16, 16)]
        pltpu.async_copy(z_hbm.at[gidx], rows, sem).wait()
        pltpu.sync_copy(rows, acc_sp.at[widx], add=True)
        return 0

    lax.fori_loop(0, NCHUNK, _chunk, 0)
    plsc.subcore_barrier()

    pltpu.sync_copy(acc_sp.at[pl.ds(s * WB, WB)],
                    out_hbm.at[c, pl.ds(s * WB, WB)])

    @pl.when(s == NS - 1)
    def _():
        pltpu.sync_copy(acc_sp.at[pl.ds(NS * WB, N - NS * WB)],
                        out_hbm.at[c, pl.ds(NS * WB, N - NS * WB)])


_aggregate = pl.kernel(
    _agg_body,
    out_type=jax.ShapeDtypeStruct((NC, N, D), jnp.float32),
    mesh=plsc.VectorSubcoreMesh(core_axis_name="c", subcore_axis_name="s"),
    scratch_types=[
        pltpu.VMEM((NCHUNK, CHUNK), jnp.int32),
        pltpu.VMEM((NCHUNK, CHUNK), jnp.int32),
        pltpu.VMEM((CHUNK,), jnp.int32),
        pltpu.VMEM((CHUNK,), jnp.int32),
        pltpu.VMEM((CHUNK, D), jnp.float32),
        pltpu.VMEM_SHARED((ROWS_PAD, D), jnp.float32),
        pltpu.SemaphoreType.DMA,
    ],
    compiler_params=pltpu.CompilerParams(needs_layout_passes=False),
)


def _cnt_body(dst_hbm, cnt_hbm, didx, cnt_v):
    c = lax.axis_index("c")
    s = lax.axis_index("s")
    wid = c * NS + s

    zeros16 = jnp.zeros((16,), jnp.float32)

    def _zero_cnt(r, _):
        cnt_v[0, pl.ds(r * 16, 16)] = zeros16
        return 0

    lax.fori_loop(0, N // 16, _zero_cnt, 0)
    pltpu.sync_copy(dst_hbm.at[wid], didx)

    ones16 = jnp.full((16,), 1.0, jnp.float32)

    def _group(g, _):
        idx = didx[0, pl.ds(g * 16, 16)]
        plsc.addupdate_scatter(cnt_v.at[0], [idx], ones16)
        return 0

    lax.fori_loop(0, EDGES_PER_TILE // 16, _group, 0)
    pltpu.sync_copy(cnt_v, cnt_hbm.at[wid])


_count_degrees = pl.kernel(
    _cnt_body,
    out_type=jax.ShapeDtypeStruct((NW, 1, N), jnp.float32),
    mesh=plsc.VectorSubcoreMesh(core_axis_name="c", subcore_axis_name="s"),
    scratch_types=[
        pltpu.VMEM((1, EDGES_PER_TILE), jnp.int32),
        pltpu.VMEM((1, N), jnp.float32),
    ],
    compiler_params=pltpu.CompilerParams(needs_layout_passes=False),
)


def kernel(x, edge_index, W1_0, b1_0, W2_0, b2_0, W1_1, b1_1, W2_1, b2_1,
           W1_2, b1_2, W2_2, b2_2):
    npad = EPT_PAD - EDGES_PER_TILE
    pad_src = jnp.zeros((NW, npad), jnp.int32)
    pad_dst = jnp.broadcast_to(
        N + (jnp.arange(npad, dtype=jnp.int32) % (ROWS_PAD - N)), (NW, npad))
    src = jnp.concatenate(
        [edge_index[0].reshape(NW, EDGES_PER_TILE), pad_src],
        axis=1).reshape(NW, NCHUNK, CHUNK)
    dst = jnp.concatenate(
        [edge_index[1].reshape(NW, EDGES_PER_TILE), pad_dst],
        axis=1).reshape(NW, NCHUNK, CHUNK)
    dstw = edge_index[1].reshape(NW, 1, EDGES_PER_TILE)
    params = [
        (W1_0, b1_0, W2_0, b2_0),
        (W1_1, b1_1, W2_1, b2_1),
        (W1_2, b1_2, W2_2, b2_2),
    ]
    cnt = _count_degrees(dstw).reshape(NW, N).T
    h = x
    for (w1, b1, w2, b2) in params:
        z = _mlp(h, w1, b1.reshape(1, D), w2, b2.reshape(1, D))
        acc = _aggregate(z, src, dst)
        h = _combine(acc, cnt, h)
    return h


# speedup vs baseline: 1.0910x; 1.0433x over previous
import jax
import jax.numpy as jnp
from jax import lax
from jax.experimental import pallas as pl
from jax.experimental.pallas import tpu as pltpu
from jax.experimental.pallas import tpu_sc as plsc

N = 10000
E = 320000
D = 128

NC = 2
NS = 16
NW = NC * NS
EDGES_PER_TILE = E // NW
CHUNK = 112
NCHUNK = 90
EPT_PAD = NCHUNK * CHUNK
ROWS_PAD = 10240
ZROWS = ROWS_PAD // NS
WB = 624

BLK = 2000
GRID = N // BLK


def _mlp_body(x_ref, w1_ref, b1_ref, w2_ref, b2_ref, o_ref):
    x = x_ref[...]
    h = jnp.dot(x, w1_ref[...], preferred_element_type=jnp.float32) + b1_ref[...]
    h = jnp.maximum(h, 0.0)
    h = jnp.dot(h, w2_ref[...], preferred_element_type=jnp.float32) + b2_ref[...]
    h = h * jax.nn.sigmoid(h)
    mu = jnp.mean(h, axis=1, keepdims=True)
    d = h - mu
    var = jnp.mean(d * d, axis=1, keepdims=True)
    o_ref[...] = d * lax.rsqrt(var + 1e-5)


_mlp = pl.pallas_call(
    _mlp_body,
    grid=(GRID,),
    in_specs=[
        pl.BlockSpec((BLK, D), lambda i: (i, 0)),
        pl.BlockSpec((D, D), lambda i: (0, 0)),
        pl.BlockSpec((1, D), lambda i: (0, 0)),
        pl.BlockSpec((D, D), lambda i: (0, 0)),
        pl.BlockSpec((1, D), lambda i: (0, 0)),
    ],
    out_specs=pl.BlockSpec((BLK, D), lambda i: (i, 0)),
    out_shape=jax.ShapeDtypeStruct((N, D), jnp.float32),
)


def _combine_body(a_ref, c_ref, hp_ref, o_ref):
    s = a_ref[0] + a_ref[1]
    cnt = jnp.sum(c_ref[...], axis=1).reshape(-1, 1)
    o_ref[...] = s / jnp.maximum(cnt, 1.0) + hp_ref[...]


_combine = pl.pallas_call(
    _combine_body,
    grid=(GRID,),
    in_specs=[
        pl.BlockSpec((2, BLK, D), lambda i: (0, i, 0)),
        pl.BlockSpec((BLK, NW), lambda i: (i, 0)),
        pl.BlockSpec((BLK, D), lambda i: (i, 0)),
    ],
    out_specs=pl.BlockSpec((BLK, D), lambda i: (i, 0)),
    out_shape=jax.ShapeDtypeStruct((N, D), jnp.float32),
)


def _agg_body(z_hbm, src_hbm, dst_hbm, out_hbm, sidx, didx, gidx, widx,
              rows, acc_sp, sem):
    c = lax.axis_index("c")
    s = lax.axis_index("s")
    wid = c * NS + s

    def _preload(g, _):
        pltpu.sync_copy(src_hbm.at[wid, pl.ds(g * 8, 8)],
                        sidx.at[pl.ds(g * 8, 8)])
        pltpu.sync_copy(dst_hbm.at[wid, pl.ds(g * 8, 8)],
                        didx.at[pl.ds(g * 8, 8)])
        return 0

    lax.fori_loop(0, NCHUNK // 8, _preload, 0)
    _tail = (NCHUNK // 8) * 8
    pltpu.sync_copy(src_hbm.at[wid, pl.ds(_tail, NCHUNK - _tail)],
                    sidx.at[pl.ds(_tail, NCHUNK - _tail)])
    pltpu.sync_copy(dst_hbm.at[wid, pl.ds(_tail, NCHUNK - _tail)],
                    didx.at[pl.ds(_tail, NCHUNK - _tail)])

    zeros16 = jnp.zeros((16,), jnp.float32)

    def _zero_row(r, _):
        for cc in range(D // 16):
            rows[r, pl.ds(cc * 16, 16)] = zeros16
        return 0

    lax.fori_loop(0, CHUNK, _zero_row, 0)
    for k in range(ZROWS // CHUNK):
        pltpu.sync_copy(rows, acc_sp.at[pl.ds(s * ZROWS + k * CHUNK, CHUNK)])
    if ZROWS % CHUNK:
        pltpu.sync_copy(rows, acc_sp.at[pl.ds(s * ZROWS + ZROWS - CHUNK,
                                              CHUNK)])
    plsc.subcore_barrier()

    def _chunk(j, _):
        for g in range(CHUNK // 16):
            gidx[pl.ds(g * 16, 16)] = sidx[j, pl.ds(g * 16, 16)]
            widx[pl.ds(g * 16, 16)] = didx[j, pl.ds(g * 16, 16)]
        pltpu.async_copy(z_hbm.at[gidx], rows, sem).wait()
        pltpu.sync_copy(rows, acc_sp.at[widx], add=True)
        return 0

    lax.fori_loop(0, NCHUNK, _chunk, 0)
    plsc.subcore_barrier()

    pltpu.sync_copy(acc_sp.at[pl.ds(s * WB, WB)],
                    out_hbm.at[c, pl.ds(s * WB, WB)])

    @pl.when(s == NS - 1)
    def _():
        pltpu.sync_copy(acc_sp.at[pl.ds(NS * WB, N - NS * WB)],
                        out_hbm.at[c, pl.ds(NS * WB, N - NS * WB)])


_aggregate = pl.kernel(
    _agg_body,
    out_type=jax.ShapeDtypeStruct((NC, N, D), jnp.float32),
    mesh=plsc.VectorSubcoreMesh(core_axis_name="c", subcore_axis_name="s"),
    scratch_types=[
        pltpu.VMEM((NCHUNK, CHUNK), jnp.int32),
        pltpu.VMEM((NCHUNK, CHUNK), jnp.int32),
        pltpu.VMEM((CHUNK,), jnp.int32),
        pltpu.VMEM((CHUNK,), jnp.int32),
        pltpu.VMEM((CHUNK, D), jnp.float32),
        pltpu.VMEM_SHARED((ROWS_PAD, D), jnp.float32),
        pltpu.SemaphoreType.DMA,
    ],
    compiler_params=pltpu.CompilerParams(needs_layout_passes=False),
)


def _cnt_body(dst_hbm, cnt_hbm, didx, cnt_v):
    c = lax.axis_index("c")
    s = lax.axis_index("s")
    wid = c * NS + s

    zeros16 = jnp.zeros((16,), jnp.float32)

    def _zero_cnt(r, _):
        cnt_v[0, pl.ds(r * 16, 16)] = zeros16
        return 0

    lax.fori_loop(0, N // 16, _zero_cnt, 0)
    pltpu.sync_copy(dst_hbm.at[wid], didx)

    ones16 = jnp.full((16,), 1.0, jnp.float32)

    def _group(g, _):
        idx = didx[0, pl.ds(g * 16, 16)]
        plsc.addupdate_scatter(cnt_v.at[0], [idx], ones16)
        return 0

    lax.fori_loop(0, EDGES_PER_TILE // 16, _group, 0)
    pltpu.sync_copy(cnt_v, cnt_hbm.at[wid])


_count_degrees = pl.kernel(
    _cnt_body,
    out_type=jax.ShapeDtypeStruct((NW, 1, N), jnp.float32),
    mesh=plsc.VectorSubcoreMesh(core_axis_name="c", subcore_axis_name="s"),
    scratch_types=[
        pltpu.VMEM((1, EDGES_PER_TILE), jnp.int32),
        pltpu.VMEM((1, N), jnp.float32),
    ],
    compiler_params=pltpu.CompilerParams(needs_layout_passes=False),
)


def kernel(x, edge_index, W1_0, b1_0, W2_0, b2_0, W1_1, b1_1, W2_1, b2_1,
           W1_2, b1_2, W2_2, b2_2):
    npad = EPT_PAD - EDGES_PER_TILE
    pad_src = jnp.zeros((NW, npad), jnp.int32)
    pad_dst = jnp.broadcast_to(
        N + (jnp.arange(npad, dtype=jnp.int32) % (ROWS_PAD - N)), (NW, npad))
    src = jnp.concatenate(
        [edge_index[0].reshape(NW, EDGES_PER_TILE), pad_src],
        axis=1).reshape(NW, NCHUNK, CHUNK)
    dst = jnp.concatenate(
        [edge_index[1].reshape(NW, EDGES_PER_TILE), pad_dst],
        axis=1).reshape(NW, NCHUNK, CHUNK)
    dstw = edge_index[1].reshape(NW, 1, EDGES_PER_TILE)
    params = [
        (W1_0, b1_0, W2_0, b2_0),
        (W1_1, b1_1, W2_1, b2_1),
        (W1_2, b1_2, W2_2, b2_2),
    ]
    cnt = _count_degrees(dstw).reshape(NW, N).T
    h = x
    for (w1, b1, w2, b2) in params:
        z = _mlp(h, w1, b1.reshape(1, D), w2, b2.reshape(1, D))
        acc = _aggregate(z, src, dst)
        h = _combine(acc, cnt, h)
    return h


# speedup vs baseline: 1.3503x; 1.2376x over previous
import jax
import jax.numpy as jnp
from jax import lax
from jax.experimental import pallas as pl
from jax.experimental.pallas import tpu as pltpu
from jax.experimental.pallas import tpu_sc as plsc

N = 10000
E = 320000
D = 128

NC = 2
NS = 16
NW = NC * NS
EDGES_PER_TILE = E // NW
CHUNK = 80
NCHUNK = EDGES_PER_TILE // CHUNK
ROWS_PAD = 10240
ZROWS = ROWS_PAD // NS
WB = 624

BLK = 2000
GRID = N // BLK


def _mlp_body(x_ref, w1_ref, b1_ref, w2_ref, b2_ref, o_ref):
    x = x_ref[...]
    h = jnp.dot(x, w1_ref[...], preferred_element_type=jnp.float32) + b1_ref[...]
    h = jnp.maximum(h, 0.0)
    h = jnp.dot(h, w2_ref[...], preferred_element_type=jnp.float32) + b2_ref[...]
    h = h * jax.nn.sigmoid(h)
    mu = jnp.mean(h, axis=1, keepdims=True)
    d = h - mu
    var = jnp.mean(d * d, axis=1, keepdims=True)
    o_ref[...] = d * lax.rsqrt(var + 1e-5)


_mlp = pl.pallas_call(
    _mlp_body,
    grid=(GRID,),
    in_specs=[
        pl.BlockSpec((BLK, D), lambda i: (i, 0)),
        pl.BlockSpec((D, D), lambda i: (0, 0)),
        pl.BlockSpec((1, D), lambda i: (0, 0)),
        pl.BlockSpec((D, D), lambda i: (0, 0)),
        pl.BlockSpec((1, D), lambda i: (0, 0)),
    ],
    out_specs=pl.BlockSpec((BLK, D), lambda i: (i, 0)),
    out_shape=jax.ShapeDtypeStruct((N, D), jnp.float32),
)


def _combine_body(a_ref, c_ref, hp_ref, o_ref):
    s = a_ref[0] + a_ref[1]
    cnt = jnp.sum(c_ref[...], axis=1).reshape(-1, 1)
    o_ref[...] = s / jnp.maximum(cnt, 1.0) + hp_ref[...]


_combine = pl.pallas_call(
    _combine_body,
    grid=(GRID,),
    in_specs=[
        pl.BlockSpec((2, BLK, D), lambda i: (0, i, 0)),
        pl.BlockSpec((BLK, NW), lambda i: (i, 0)),
        pl.BlockSpec((BLK, D), lambda i: (i, 0)),
    ],
    out_specs=pl.BlockSpec((BLK, D), lambda i: (i, 0)),
    out_shape=jax.ShapeDtypeStruct((N, D), jnp.float32),
)


def _agg_body(z_hbm, src_hbm, dst_hbm, out_hbm, sidx, didx, gidx, widx,
              rows, acc_sp, sem):
    c = lax.axis_index("c")
    s = lax.axis_index("s")
    wid = c * NS + s

    def _preload(g, _):
        pltpu.sync_copy(src_hbm.at[wid, pl.ds(g * 8, 8)],
                        sidx.at[pl.ds(g * 8, 8)])
        pltpu.sync_copy(dst_hbm.at[wid, pl.ds(g * 8, 8)],
                        didx.at[pl.ds(g * 8, 8)])
        return 0

    lax.fori_loop(0, NCHUNK // 8, _preload, 0)
    _tail = (NCHUNK // 8) * 8
    pltpu.sync_copy(src_hbm.at[wid, pl.ds(_tail, NCHUNK - _tail)],
                    sidx.at[pl.ds(_tail, NCHUNK - _tail)])
    pltpu.sync_copy(dst_hbm.at[wid, pl.ds(_tail, NCHUNK - _tail)],
                    didx.at[pl.ds(_tail, NCHUNK - _tail)])

    zeros16 = jnp.zeros((16,), jnp.float32)

    def _zero_row(r, _):
        for cc in range(D // 16):
            rows[r, pl.ds(cc * 16, 16)] = zeros16
        return 0

    lax.fori_loop(0, CHUNK, _zero_row, 0)
    for k in range(ZROWS // CHUNK):
        pltpu.sync_copy(rows, acc_sp.at[pl.ds(s * ZROWS + k * CHUNK, CHUNK)])
    if ZROWS % CHUNK:
        pltpu.sync_copy(rows, acc_sp.at[pl.ds(s * ZROWS + ZROWS - CHUNK,
                                              CHUNK)])
    plsc.subcore_barrier()

    def _chunk(j, _):
        for g in range(CHUNK // 16):
            gidx[pl.ds(g * 16, 16)] = sidx[j, pl.ds(g * 16, 16)]
            widx[pl.ds(g * 16, 16)] = didx[j, pl.ds(g * 16, 16)]
        pltpu.async_copy(z_hbm.at[gidx], rows, sem).wait()
        pltpu.sync_copy(rows, acc_sp.at[widx], add=True)
        return 0

    lax.fori_loop(0, NCHUNK, _chunk, 0)
    plsc.subcore_barrier()

    pltpu.sync_copy(acc_sp.at[pl.ds(s * WB, WB)],
                    out_hbm.at[c, pl.ds(s * WB, WB)])

    @pl.when(s == NS - 1)
    def _():
        pltpu.sync_copy(acc_sp.at[pl.ds(NS * WB, N - NS * WB)],
                        out_hbm.at[c, pl.ds(NS * WB, N - NS * WB)])


_aggregate = pl.kernel(
    _agg_body,
    out_type=jax.ShapeDtypeStruct((NC, N, D), jnp.float32),
    mesh=plsc.VectorSubcoreMesh(core_axis_name="c", subcore_axis_name="s"),
    scratch_types=[
        pltpu.VMEM((NCHUNK, CHUNK), jnp.int32),
        pltpu.VMEM((NCHUNK, CHUNK), jnp.int32),
        pltpu.VMEM((CHUNK,), jnp.int32),
        pltpu.VMEM((CHUNK,), jnp.int32),
        pltpu.VMEM((CHUNK, D), jnp.float32),
        pltpu.VMEM_SHARED((ROWS_PAD, D), jnp.float32),
        pltpu.SemaphoreType.DMA,
    ],
    compiler_params=pltpu.CompilerParams(needs_layout_passes=False),
)


def _cnt_body(dst_hbm, cnt_hbm, didx, cnt_v):
    c = lax.axis_index("c")
    s = lax.axis_index("s")
    wid = c * NS + s

    zeros16 = jnp.zeros((16,), jnp.float32)

    def _zero_cnt(r, _):
        cnt_v[0, pl.ds(r * 16, 16)] = zeros16
        return 0

    lax.fori_loop(0, N // 16, _zero_cnt, 0)
    pltpu.sync_copy(dst_hbm.at[wid], didx)

    ones16 = jnp.full((16,), 1.0, jnp.float32)

    def _group(g, _):
        idx = didx[0, pl.ds(g * 16, 16)]
        plsc.addupdate_scatter(cnt_v.at[0], [idx], ones16)
        return 0

    lax.fori_loop(0, EDGES_PER_TILE // 16, _group, 0)
    pltpu.sync_copy(cnt_v, cnt_hbm.at[wid])


_count_degrees = pl.kernel(
    _cnt_body,
    out_type=jax.ShapeDtypeStruct((NW, 1, N), jnp.float32),
    mesh=plsc.VectorSubcoreMesh(core_axis_name="c", subcore_axis_name="s"),
    scratch_types=[
        pltpu.VMEM((1, EDGES_PER_TILE), jnp.int32),
        pltpu.VMEM((1, N), jnp.float32),
    ],
    compiler_params=pltpu.CompilerParams(needs_layout_passes=False),
)


def kernel(x, edge_index, W1_0, b1_0, W2_0, b2_0, W1_1, b1_1, W2_1, b2_1,
           W1_2, b1_2, W2_2, b2_2):
    src = edge_index[0].reshape(NW, NCHUNK, CHUNK)
    dst = edge_index[1].reshape(NW, NCHUNK, CHUNK)
    dstw = edge_index[1].reshape(NW, 1, EDGES_PER_TILE)
    params = [
        (W1_0, b1_0, W2_0, b2_0),
        (W1_1, b1_1, W2_1, b2_1),
        (W1_2, b1_2, W2_2, b2_2),
    ]
    cnt = _count_degrees(dstw).reshape(NW, N).T
    h = x
    for (w1, b1, w2, b2) in params:
        z = _mlp(h, w1, b1.reshape(1, D), w2, b2.reshape(1, D))
        acc = _aggregate(z, src, dst)
        h = _combine(acc, cnt, h)
    return h
